# baseline (speedup 1.0000x reference)
import functools

import jax
import jax.numpy as jnp
from jax import lax
from jax.experimental import pallas as pl
from jax.experimental.pallas import tpu as pltpu
from jax.experimental.pallas import tpu_sc as plsc


def _transform_body(w_ref, table_ref, b_ref, out_ref):
    out_ref[...] = (
        lax.dot_general(
            w_ref[...],
            table_ref[...],
            (((1,), (1,)), ((), ())),
            preferred_element_type=jnp.float32,
        )
        + b_ref[...]
    )


def _make_gather(seq: int, n_b: int, d: int, v_pad: int):
    mesh = plsc.VectorSubcoreMesh(core_axis_name="c", subcore_axis_name="s")
    nw = mesh.num_cores * mesh.num_subcores
    assert n_b % nw == 0 and seq % 2 == 0
    b_per_w = n_b // nw

    nbuf = 1
    bc = b_per_w // nbuf
    nslot = 2
    n_chunks = seq * nbuf
    assert n_chunks % nslot == 0 and bc % 16 == 0

    @functools.partial(
        pl.kernel,
        mesh=mesh,
        out_type=jax.ShapeDtypeStruct((seq, d, n_b), jnp.float32),
        scratch_types=[
            pltpu.VMEM((d * v_pad,), jnp.float32),
            pltpu.VMEM((nslot * bc,), jnp.int32),
            [pltpu.VMEM((d, bc), jnp.float32)] * nslot,
            [pltpu.SemaphoreType.DMA] * nslot,
            [pltpu.SemaphoreType.DMA] * nslot,
        ],
        compiler_params=pltpu.CompilerParams(needs_layout_passes=False),
    )
    def gather(t2t_hbm, idx_hbm, out_hbm, t2t_v, idx_v, slabs, isems, osems):
        wid = lax.axis_index("s") * mesh.num_cores + lax.axis_index("c")
        b0 = wid * b_per_w
        pltpu.sync_copy(t2t_hbm, t2t_v)

        def chunk_coords(cc):
            l = cc // nbuf
            boff = b0 + (cc % nbuf) * bc
            return l, boff

        def idx_start(cc, slot):
            l, boff = chunk_coords(cc)
            pltpu.async_copy(
                idx_hbm.at[l, pl.ds(boff, bc)],
                idx_v.at[pl.ds(slot * bc, bc)], isems[slot])

        def idx_wait(slot):
            pltpu.make_async_copy(
                idx_hbm.at[0, pl.ds(b0, bc)],
                idx_v.at[pl.ds(slot * bc, bc)], isems[slot]).wait()

        for s in range(nslot - 1):
            idx_start(s, s)

        @pl.loop(0, n_chunks, step=nslot)
        def _ring(q):
            for s in range(nslot):
                cc = q + s
                l, boff = chunk_coords(cc)
                out_slice = out_hbm.at[l, :, pl.ds(boff, bc)]

                @pl.when(cc + nslot - 1 < n_chunks)
                def _prefetch():
                    idx_start(cc + nslot - 1, (s + nslot - 1) % nslot)

                idx_wait(s)

                @pl.when(cc >= nslot)
                def _drain():
                    pltpu.make_async_copy(slabs[s], out_slice, osems[s]).wait()

                @plsc.parallel_loop(0, bc // 16, unroll=3)
                def _grp(g):
                    ids16 = idx_v[pl.ds(s * bc + g * 16, 16)]
                    for o in range(d):
                        vals = plsc.load_gather(t2t_v, [ids16 + o * v_pad])
                        slabs[s][o, pl.ds(g * 16, 16)] = vals

                pltpu.async_copy(slabs[s], out_slice, osems[s])

        for s in range(nslot):
            pltpu.make_async_copy(
                slabs[s], out_hbm.at[0, :, pl.ds(b0, bc)], osems[s]).wait()

    return gather


def kernel(input_ids, table, W, b):
    bsz, seq = input_ids.shape
    v, d = table.shape

    v_pad = 128
    table_pad = jnp.zeros((v_pad, d), jnp.float32).at[:v].set(table)

    t2t = pl.pallas_call(
        _transform_body,
        out_shape=jax.ShapeDtypeStruct((d, v_pad), jnp.float32),
    )(W, table_pad, jnp.reshape(b, (d, 1)))

    ids_t = input_ids.T.astype(jnp.int32)
    out_lob = _make_gather(seq, bsz, d, v_pad)(t2t.reshape(d * v_pad), ids_t)
    return jnp.transpose(out_lob, (2, 0, 1))


# speedup vs baseline: 1.5621x; 1.5621x over previous
import functools

import jax
import jax.numpy as jnp
from jax import lax
from jax.experimental import pallas as pl
from jax.experimental.pallas import tpu as pltpu
from jax.experimental.pallas import tpu_sc as plsc


def _transform_body(w_ref, table_ref, b_ref, out_ref):
    out_ref[...] = (
        lax.dot_general(
            w_ref[...],
            table_ref[...],
            (((1,), (1,)), ((), ())),
            preferred_element_type=jnp.float32,
        )
        + b_ref[...]
    )


def _make_gather(seq: int, n_b: int, d: int, v_pad: int):
    mesh = plsc.VectorSubcoreMesh(core_axis_name="c", subcore_axis_name="s")
    nw = mesh.num_cores * mesh.num_subcores
    assert n_b % nw == 0 and seq % 2 == 0
    b_per_w = n_b // nw

    nbuf = 1
    bc = b_per_w // nbuf
    nslot = 2
    n_chunks = seq * nbuf
    assert n_chunks % nslot == 0 and bc % 16 == 0

    @functools.partial(
        pl.kernel,
        mesh=mesh,
        out_type=jax.ShapeDtypeStruct((seq, d, n_b), jnp.float32),
        scratch_types=[
            pltpu.VMEM((d * v_pad,), jnp.float32),
            pltpu.VMEM((nslot * bc,), jnp.int32),
            [pltpu.VMEM((d, bc), jnp.float32)] * nslot,
            [pltpu.SemaphoreType.DMA] * nslot,
            [pltpu.SemaphoreType.DMA] * nslot,
        ],
        compiler_params=pltpu.CompilerParams(needs_layout_passes=False),
    )
    def gather(t2t_hbm, idx_hbm, out_hbm, t2t_v, idx_v, slabs, isems, osems):
        wid = lax.axis_index("s") * mesh.num_cores + lax.axis_index("c")
        b0 = wid * b_per_w
        pltpu.sync_copy(t2t_hbm, t2t_v)

        def chunk_coords(cc):
            l = cc // nbuf
            boff = b0 + (cc % nbuf) * bc
            return l, boff

        def idx_start(cc, slot):
            l, boff = chunk_coords(cc)
            pltpu.async_copy(
                idx_hbm.at[l, pl.ds(boff, bc)],
                idx_v.at[pl.ds(slot * bc, bc)], isems[slot])

        def idx_wait(slot):
            pltpu.make_async_copy(
                idx_hbm.at[0, pl.ds(b0, bc)],
                idx_v.at[pl.ds(slot * bc, bc)], isems[slot]).wait()

        for s in range(nslot - 1):
            idx_start(s, s)

        @pl.loop(0, n_chunks, step=nslot)
        def _ring(q):
            for s in range(nslot):
                cc = q + s
                l, boff = chunk_coords(cc)
                out_slice = out_hbm.at[l, :, pl.ds(boff, bc)]

                @pl.when(cc + nslot - 1 < n_chunks)
                def _prefetch():
                    idx_start(cc + nslot - 1, (s + nslot - 1) % nslot)

                idx_wait(s)

                @pl.when(cc >= nslot)
                def _drain():
                    pltpu.make_async_copy(slabs[s], out_slice, osems[s]).wait()

                @plsc.parallel_loop(0, bc // 16, unroll=4)
                def _grp(g):
                    ids16 = idx_v[pl.ds(s * bc + g * 16, 16)]
                    for o in range(d):
                        vals = plsc.load_gather(t2t_v, [ids16 + o * v_pad])
                        slabs[s][o, pl.ds(g * 16, 16)] = vals

                pltpu.async_copy(slabs[s], out_slice, osems[s])

        for s in range(nslot):
            pltpu.make_async_copy(
                slabs[s], out_hbm.at[0, :, pl.ds(b0, bc)], osems[s]).wait()

    return gather


def kernel(input_ids, table, W, b):
    bsz, seq = input_ids.shape
    v, d = table.shape

    v_pad = 128
    table_pad = jnp.zeros((v_pad, d), jnp.float32).at[:v].set(table)

    t2t = pl.pallas_call(
        _transform_body,
        out_shape=jax.ShapeDtypeStruct((d, v_pad), jnp.float32),
    )(W, table_pad, jnp.reshape(b, (d, 1)))

    ids_t = input_ids.T.astype(jnp.int32)
    out_lob = _make_gather(seq, bsz, d, v_pad)(t2t.reshape(d * v_pad), ids_t)
    return jnp.transpose(out_lob, (2, 0, 1))
